# baseline (speedup 1.0000x reference)
import functools

import jax
import jax.numpy as jnp
from jax import lax
from jax.experimental import pallas as pl
from jax.experimental.pallas import tpu as pltpu
from jax.experimental.pallas import tpu_sc as plsc

N = 10000
E = 320000
DF = 128
DE = 16
TE = 3
TN = 2

NW = 32
CHUNK = 128
NCHUNK = 80
PW = CHUNK * NCHUNK
EP = NW * PW
ROWS = EP * DE // 128
ROWS_REAL = E * DE // 128

_HIGH = lax.Precision.HIGHEST


def _dot(a, b):
    return lax.dot_general(a, b, (((1,), (0,)), ((), ())),
                           precision=_HIGH, preferred_element_type=jnp.float32)


def _proj_body(nf_ref, w_ref, b_ref, o_ref):
    o_ref[...] = _dot(nf_ref[...], w_ref[...]) + b_ref[0:1, :]


def _proj(nf, wpad, bpad):
    return pl.pallas_call(
        _proj_body,
        grid=(5,),
        in_specs=[pl.BlockSpec((2000, DF), lambda i: (i, 0)),
                  pl.BlockSpec((DF, 96), lambda i: (0, 0)),
                  pl.BlockSpec((8, 96), lambda i: (0, 0))],
        out_specs=pl.BlockSpec((2000, 96), lambda i: (i, 0)),
        out_shape=jax.ShapeDtypeStruct((N, 96), jnp.float32),
    )(nf, wpad, bpad)


MB = 512


def _efp_body(ef_ref, et_ref, wbd_ref, o_ref):
    x = ef_ref[...]
    et = et_ref[...]
    acc = jnp.where(et == 0, _dot(x, wbd_ref[0]), 0.0)
    for t in range(1, TE):
        acc += jnp.where(et == t, _dot(x, wbd_ref[t]), 0.0)
    o_ref[...] = acc


def _efp(ef_rs, et_rep, wbd):
    return pl.pallas_call(
        _efp_body,
        grid=(ROWS // MB,),
        in_specs=[pl.BlockSpec((MB, 128), lambda i: (i, 0)),
                  pl.BlockSpec((MB, 128), lambda i: (i, 0)),
                  pl.BlockSpec((TE, 128, 128), lambda i: (0, 0, 0))],
        out_specs=pl.BlockSpec((MB, 128), lambda i: (i, 0)),
        out_shape=jax.ShapeDtypeStruct((ROWS, 128), jnp.float32),
    )(ef_rs, et_rep, wbd)


G = 4
GROUP = G * CHUNK
NG = PW // GROUP
EROW = GROUP * DE // 128
NCH = E // CHUNK
KMAX = (NCH + NW - 1) // NW


def _sc_edge_body(p_hbm, is_hbm, id_hbm, efp_hbm, uef_hbm,
                  iv, rv, ev, p_sh, sems0, semd0, sems1, semd1):
    semS = (sems0, sems1)
    semD = (semd0, semd1)
    cid = lax.axis_index("c")
    sid = lax.axis_index("s")
    wid = sid * 2 + cid

    if True:
        @pl.when(sid == 0)
        def _():
            pltpu.sync_copy(p_hbm, p_sh)

        plsc.subcore_barrier()

        base = wid * PW
        brow = wid * (PW // CHUNK)
        erow = wid * (PW * DE // 128)

        def load(g, sl):
            pltpu.sync_copy(is_hbm.at[pl.ds(brow + g * G, G)], iv.at[sl, 0])
            pltpu.sync_copy(id_hbm.at[pl.ds(brow + g * G, G)], iv.at[sl, 1])
            pltpu.sync_copy(efp_hbm.at[pl.ds(erow + g * EROW, EROW)], ev.at[sl])

        def fire(sl):
            for j in range(G):
                pltpu.async_copy(p_sh.at[iv.at[sl, 0, j]],
                                 rv.at[sl, 0, pl.ds(j * CHUNK, CHUNK)],
                                 semS[sl])
                pltpu.async_copy(p_sh.at[iv.at[sl, 1, j]],
                                 rv.at[sl, 1, pl.ds(j * CHUNK, CHUNK)],
                                 semD[sl])

        def wait(sl):
            for j in range(G):
                pltpu.make_async_copy(p_sh.at[iv.at[sl, 0, j]],
                                      rv.at[sl, 0, pl.ds(j * CHUNK, CHUNK)],
                                      semS[sl]).wait()
                pltpu.make_async_copy(p_sh.at[iv.at[sl, 1, j]],
                                      rv.at[sl, 1, pl.ds(j * CHUNK, CHUNK)],
                                      semD[sl]).wait()

        def compute(sl):
            @pl.loop(0, EROW)
            def _(j):
                for k in range(128 // DE):
                    e = j * (128 // DE) + k
                    v = (rv[sl, 0, e, :] + rv[sl, 1, e, :]
                         + ev[sl, j, pl.ds(k * DE, DE)])
                    rv[sl, 0, e, :] = jnp.maximum(v, 0.0)

        def store(g, sl):
            pltpu.sync_copy(rv.at[sl, 0],
                            uef_hbm.at[pl.ds(base + g * GROUP, GROUP)])

        load(0, 0)
        fire(0)

        @pl.loop(0, NG, step=2)
        def _(g):
            load(g + 1, 1)
            fire(1)
            wait(0)
            compute(0)
            store(g, 0)

            @pl.when(g + 2 < NG)
            def _():
                load(g + 2, 0)
                fire(0)

            wait(1)
            compute(1)
            store(g + 1, 1)


def _sc_scatter_body(uef_hbm, didx_hbm, zero_hbm, part_hbm,
                     vv, iv2, agg_sh, seml0, seml1):
    cid = lax.axis_index("c")
    sid = lax.axis_index("s")
    wid = sid * 2 + cid

    if True:
        @pl.when(sid == 0)
        def _():
            pltpu.sync_copy(zero_hbm, agg_sh)

        plsc.subcore_barrier()

        sems2 = (seml0, seml1)

        def aload(k, sl):
            off = (wid + k * NW) * CHUNK
            pltpu.async_copy(uef_hbm.at[pl.ds(off, CHUNK)], vv.at[sl],
                             sems2[sl])
            pltpu.async_copy(didx_hbm.at[pl.ds(off, CHUNK)], iv2.at[sl],
                             sems2[sl])

        def wload(k, sl):
            off = (wid + k * NW) * CHUNK
            pltpu.make_async_copy(uef_hbm.at[pl.ds(off, CHUNK)], vv.at[sl],
                                  sems2[sl]).wait()
            pltpu.make_async_copy(didx_hbm.at[pl.ds(off, CHUNK)], iv2.at[sl],
                                  sems2[sl]).wait()

        def sadd(sl):
            pltpu.sync_copy(vv.at[sl], agg_sh.at[iv2.at[sl]], add=True)

        nk = 78 + jnp.where(wid + 78 * NW < NCH, 1, 0)

        aload(0, 0)

        @pl.loop(0, KMAX, step=2)
        def _(k):
            @pl.when(k < nk)
            def _():
                @pl.when(k + 1 < nk)
                def _():
                    aload(k + 1, 1)

                wload(k, 0)
                sadd(0)

                @pl.when(k + 2 < nk)
                def _():
                    aload(k + 2, 0)

                @pl.when(k + 1 < nk)
                def _():
                    wload(k + 1, 1)
                    sadd(1)

        plsc.subcore_barrier()
        rows = N // 16
        pltpu.sync_copy(agg_sh.at[pl.ds(sid * rows, rows)],
                        part_hbm.at[cid, pl.ds(sid * rows, rows)])


def _sc_edge(p8, idx_s2, idx_d2, efp):
    mesh = plsc.VectorSubcoreMesh(core_axis_name="c", subcore_axis_name="s")
    f = pl.kernel(
        _sc_edge_body,
        mesh=mesh,
        compiler_params=pltpu.CompilerParams(use_tc_tiling_on_sc=False),
        out_type=jax.ShapeDtypeStruct((EP, DE), jnp.float32),
        scratch_types=[pltpu.VMEM((2, 2, G, CHUNK), jnp.int32),
                       pltpu.VMEM((2, 2, GROUP, DE), jnp.float32),
                       pltpu.VMEM((2, EROW, 128), jnp.float32),
                       pltpu.VMEM_SHARED((N * 6, DE), jnp.float32),
                       pltpu.SemaphoreType.DMA,
                       pltpu.SemaphoreType.DMA,
                       pltpu.SemaphoreType.DMA,
                       pltpu.SemaphoreType.DMA],
    )
    return f(p8, idx_s2, idx_d2, efp)


def _sc_scatter(uef, dst, zeros_n):
    mesh = plsc.VectorSubcoreMesh(core_axis_name="c", subcore_axis_name="s")
    f = pl.kernel(
        _sc_scatter_body,
        mesh=mesh,
        compiler_params=pltpu.CompilerParams(use_tc_tiling_on_sc=False),
        out_type=jax.ShapeDtypeStruct((2, N, DE), jnp.float32),
        scratch_types=[pltpu.VMEM((2, CHUNK, DE), jnp.float32),
                       pltpu.VMEM((2, CHUNK), jnp.int32),
                       pltpu.VMEM_SHARED((N, DE), jnp.float32),
                       pltpu.SemaphoreType.DMA,
                       pltpu.SemaphoreType.DMA],
    )
    return f(uef, dst, zeros_n)


def _node_body(part_ref, nf_ref, nt_ref, wa_ref, wb_ref, bn_ref, o_ref):
    agg = part_ref[0] + part_ref[1]
    x = nf_ref[...]
    nt = nt_ref[...]
    y0 = jnp.maximum(_dot(agg, wa_ref[0]) + _dot(x, wb_ref[0]) + bn_ref[0], 0.0)
    y1 = jnp.maximum(_dot(agg, wa_ref[1]) + _dot(x, wb_ref[1]) + bn_ref[1], 0.0)
    o_ref[...] = jnp.where(nt == 0, y0, y1)


def _node(part, nf, ntype2, wa, wb, bn_pad):
    return pl.pallas_call(
        _node_body,
        grid=(5,),
        in_specs=[pl.BlockSpec((2, 2000, DE), lambda i: (0, i, 0)),
                  pl.BlockSpec((2000, DF), lambda i: (i, 0)),
                  pl.BlockSpec((2000, 1), lambda i: (i, 0)),
                  pl.BlockSpec((TN, DE, DF), lambda i: (0, 0, 0)),
                  pl.BlockSpec((TN, DF, DF), lambda i: (0, 0, 0)),
                  pl.BlockSpec((8, DF), lambda i: (0, 0))],
        out_specs=pl.BlockSpec((2000, DF), lambda i: (i, 0)),
        out_shape=jax.ShapeDtypeStruct((N, DF), jnp.float32),
    )(part, nf, ntype2, wa, wb, bn_pad)


def kernel(nf, ef, edge_index, etype, ntype, We, be, Wn, bn):
    src = edge_index[0]
    dst = edge_index[1]

    ws = jnp.transpose(We[:, DE:DE + DF, :], (1, 0, 2)).reshape(DF, TE * DE)
    wd = jnp.transpose(We[:, DE + DF:, :], (1, 0, 2)).reshape(DF, TE * DE)
    wpad = jnp.concatenate([ws, wd], axis=1)
    eye8 = jnp.eye(8, dtype=jnp.float32)
    wbd = jax.vmap(lambda w: jnp.kron(eye8, w))(We[:, :DE, :])
    bpad = jnp.tile(jnp.concatenate(
        [jnp.zeros((3, DE), jnp.float32), be], axis=0).reshape(1, 96), (8, 1))
    wa = Wn[:, :DE, :]
    wb = Wn[:, DE:, :]
    bn_pad = jnp.concatenate(
        [bn, jnp.zeros((8 - TN, DF), jnp.float32)], axis=0)

    pad = EP - E
    idx_s2 = jnp.pad(src * 6 + etype, (0, pad)).reshape(EP // CHUNK, CHUNK)
    idx_d2 = jnp.pad(dst * 6 + 3 + etype, (0, pad)).reshape(EP // CHUNK, CHUNK)
    ef_rs = jnp.pad(ef, ((0, pad), (0, 0))).reshape(ROWS, 128)
    et_rep = jnp.repeat(jnp.pad(etype, (0, pad)), DE).reshape(ROWS, 128)
    ntype2 = ntype.reshape(N, 1)
    zeros_n = jnp.zeros((N, DE), jnp.float32)

    p = _proj(nf, wpad, bpad)
    p8 = p.reshape(N * 6, DE)
    efp = _efp(ef_rs, et_rep, wbd)

    uef = _sc_edge(p8, idx_s2, idx_d2, efp)
    part = _sc_scatter(uef, dst, zeros_n)

    updated_nf = _node(part, nf, ntype2, wa, wb, bn_pad)

    return (updated_nf, uef[:E])


# speedup vs baseline: 1.6380x; 1.6380x over previous
import functools

import jax
import jax.numpy as jnp
from jax import lax
from jax.experimental import pallas as pl
from jax.experimental.pallas import tpu as pltpu
from jax.experimental.pallas import tpu_sc as plsc

N = 10000
E = 320000
DF = 128
DE = 16
TE = 3
TN = 2

NW = 32
CHUNK = 128
NCHUNK = 80
PW = CHUNK * NCHUNK
EP = NW * PW
ROWS = EP * DE // 128
ROWS_REAL = E * DE // 128

_HIGH = lax.Precision.HIGHEST


def _dot(a, b):
    return lax.dot_general(a, b, (((1,), (0,)), ((), ())),
                           precision=_HIGH, preferred_element_type=jnp.float32)


def _proj_body(nf_ref, w_ref, b_ref, o_ref):
    o_ref[...] = _dot(nf_ref[...], w_ref[...]) + b_ref[0:1, :]


def _proj(nf, wpad, bpad):
    return pl.pallas_call(
        _proj_body,
        grid=(5,),
        in_specs=[pl.BlockSpec((2000, DF), lambda i: (i, 0)),
                  pl.BlockSpec((DF, 96), lambda i: (0, 0)),
                  pl.BlockSpec((8, 96), lambda i: (0, 0))],
        out_specs=pl.BlockSpec((2000, 96), lambda i: (i, 0)),
        out_shape=jax.ShapeDtypeStruct((N, 96), jnp.float32),
    )(nf, wpad, bpad)


MB = 800


def _msg_body(ef_ref, et_ref, s_ref, wbd_ref, exp_ref, o_ref):
    x = ef_ref[...]
    et8 = et_ref[...]
    acc = s_ref[...]
    for t in range(TE):
        mask = _dot((et8 == t).astype(jnp.float32), exp_ref[...])
        acc += mask * _dot(x, wbd_ref[t])
    o_ref[...] = jnp.maximum(acc, 0.0)


def _msg(ef_rs, et8, s_rs, wbd, exp8):
    return pl.pallas_call(
        _msg_body,
        grid=(ROWS_REAL // MB,),
        in_specs=[pl.BlockSpec((MB, 128), lambda i: (i, 0)),
                  pl.BlockSpec((MB, 8), lambda i: (i, 0)),
                  pl.BlockSpec((MB, 128), lambda i: (i, 0)),
                  pl.BlockSpec((TE, 128, 128), lambda i: (0, 0, 0)),
                  pl.BlockSpec((8, 128), lambda i: (0, 0))],
        out_specs=pl.BlockSpec((MB, 128), lambda i: (i, 0)),
        out_shape=jax.ShapeDtypeStruct((ROWS_REAL, 128), jnp.float32),
    )(ef_rs, et8, s_rs, wbd, exp8)


G = 8
GROUP = G * CHUNK
NG = PW // GROUP
NCH = E // CHUNK
KMAX = (NCH + NW - 1) // NW


def _sc_edge_body(p_hbm, is_hbm, id_hbm, s_out_hbm,
                  iv, rv, p_sh, sems0, semd0, sems1, semd1):
    semS = (sems0, sems1)
    semD = (semd0, semd1)
    cid = lax.axis_index("c")
    sid = lax.axis_index("s")
    wid = sid * 2 + cid

    if True:
        @pl.when(sid == 0)
        def _():
            pltpu.sync_copy(p_hbm, p_sh)

        plsc.subcore_barrier()

        base = wid * PW
        brow = wid * (PW // CHUNK)

        def load(g, sl):
            pltpu.sync_copy(is_hbm.at[pl.ds(brow + g * G, G)], iv.at[sl, 0])
            pltpu.sync_copy(id_hbm.at[pl.ds(brow + g * G, G)], iv.at[sl, 1])

        def fire(sl):
            for j in range(G):
                pltpu.async_copy(p_sh.at[iv.at[sl, 0, j]],
                                 rv.at[sl, 0, pl.ds(j * CHUNK, CHUNK)],
                                 semS[sl])
                pltpu.async_copy(p_sh.at[iv.at[sl, 1, j]],
                                 rv.at[sl, 1, pl.ds(j * CHUNK, CHUNK)],
                                 semD[sl])

        def wait(sl):
            for j in range(G):
                pltpu.make_async_copy(p_sh.at[iv.at[sl, 0, j]],
                                      rv.at[sl, 0, pl.ds(j * CHUNK, CHUNK)],
                                      semS[sl]).wait()
                pltpu.make_async_copy(p_sh.at[iv.at[sl, 1, j]],
                                      rv.at[sl, 1, pl.ds(j * CHUNK, CHUNK)],
                                      semD[sl]).wait()

        def compute(sl):
            @pl.loop(0, GROUP)
            def _(e):
                rv[sl, 0, e, :] = rv[sl, 0, e, :] + rv[sl, 1, e, :]

        def store(g, sl):
            pltpu.sync_copy(rv.at[sl, 0],
                            s_out_hbm.at[pl.ds(base + g * GROUP, GROUP)])

        load(0, 0)
        fire(0)

        @pl.loop(0, NG, step=2)
        def _(g):
            load(g + 1, 1)
            fire(1)
            wait(0)
            compute(0)
            store(g, 0)

            @pl.when(g + 2 < NG)
            def _():
                load(g + 2, 0)
                fire(0)

            wait(1)
            compute(1)
            store(g + 1, 1)


def _sc_scatter_body(uef_hbm, didx_hbm, zero_hbm, part_hbm,
                     vv, iv2, agg_sh, seml0, seml1):
    cid = lax.axis_index("c")
    sid = lax.axis_index("s")
    wid = sid * 2 + cid

    if True:
        @pl.when(sid == 0)
        def _():
            pltpu.sync_copy(zero_hbm, agg_sh)

        plsc.subcore_barrier()

        sems2 = (seml0, seml1)

        def aload(k, sl):
            off = (wid + k * NW) * CHUNK
            pltpu.async_copy(uef_hbm.at[pl.ds(off, CHUNK)], vv.at[sl],
                             sems2[sl])
            pltpu.async_copy(didx_hbm.at[pl.ds(off, CHUNK)], iv2.at[sl],
                             sems2[sl])

        def wload(k, sl):
            off = (wid + k * NW) * CHUNK
            pltpu.make_async_copy(uef_hbm.at[pl.ds(off, CHUNK)], vv.at[sl],
                                  sems2[sl]).wait()
            pltpu.make_async_copy(didx_hbm.at[pl.ds(off, CHUNK)], iv2.at[sl],
                                  sems2[sl]).wait()

        def sadd(sl):
            pltpu.sync_copy(vv.at[sl], agg_sh.at[iv2.at[sl]], add=True)

        nk = 78 + jnp.where(wid + 78 * NW < NCH, 1, 0)

        aload(0, 0)

        @pl.loop(0, KMAX, step=2)
        def _(k):
            @pl.when(k < nk)
            def _():
                @pl.when(k + 1 < nk)
                def _():
                    aload(k + 1, 1)

                wload(k, 0)
                sadd(0)

                @pl.when(k + 2 < nk)
                def _():
                    aload(k + 2, 0)

                @pl.when(k + 1 < nk)
                def _():
                    wload(k + 1, 1)
                    sadd(1)

        plsc.subcore_barrier()
        rows = N // 16
        pltpu.sync_copy(agg_sh.at[pl.ds(sid * rows, rows)],
                        part_hbm.at[cid, pl.ds(sid * rows, rows)])


def _sc_edge(p8, idx_s2, idx_d2):
    mesh = plsc.VectorSubcoreMesh(core_axis_name="c", subcore_axis_name="s")
    f = pl.kernel(
        _sc_edge_body,
        mesh=mesh,
        compiler_params=pltpu.CompilerParams(use_tc_tiling_on_sc=False),
        out_type=jax.ShapeDtypeStruct((EP, DE), jnp.float32),
        scratch_types=[pltpu.VMEM((2, 2, G, CHUNK), jnp.int32),
                       pltpu.VMEM((2, 2, GROUP, DE), jnp.float32),
                       pltpu.VMEM_SHARED((N * 6, DE), jnp.float32),
                       pltpu.SemaphoreType.DMA,
                       pltpu.SemaphoreType.DMA,
                       pltpu.SemaphoreType.DMA,
                       pltpu.SemaphoreType.DMA],
    )
    return f(p8, idx_s2, idx_d2)


def _sc_scatter(uef, dst, zeros_n):
    mesh = plsc.VectorSubcoreMesh(core_axis_name="c", subcore_axis_name="s")
    f = pl.kernel(
        _sc_scatter_body,
        mesh=mesh,
        compiler_params=pltpu.CompilerParams(use_tc_tiling_on_sc=False),
        out_type=jax.ShapeDtypeStruct((2, N, DE), jnp.float32),
        scratch_types=[pltpu.VMEM((2, CHUNK, DE), jnp.float32),
                       pltpu.VMEM((2, CHUNK), jnp.int32),
                       pltpu.VMEM_SHARED((N, DE), jnp.float32),
                       pltpu.SemaphoreType.DMA,
                       pltpu.SemaphoreType.DMA],
    )
    return f(uef, dst, zeros_n)


def _node_body(part_ref, nf_ref, nt_ref, wa_ref, wb_ref, bn_ref, o_ref):
    agg = part_ref[0] + part_ref[1]
    x = nf_ref[...]
    nt = nt_ref[...]
    y0 = jnp.maximum(_dot(agg, wa_ref[0]) + _dot(x, wb_ref[0]) + bn_ref[0], 0.0)
    y1 = jnp.maximum(_dot(agg, wa_ref[1]) + _dot(x, wb_ref[1]) + bn_ref[1], 0.0)
    o_ref[...] = jnp.where(nt == 0, y0, y1)


def _node(part, nf, ntype2, wa, wb, bn_pad):
    return pl.pallas_call(
        _node_body,
        grid=(5,),
        in_specs=[pl.BlockSpec((2, 2000, DE), lambda i: (0, i, 0)),
                  pl.BlockSpec((2000, DF), lambda i: (i, 0)),
                  pl.BlockSpec((2000, 1), lambda i: (i, 0)),
                  pl.BlockSpec((TN, DE, DF), lambda i: (0, 0, 0)),
                  pl.BlockSpec((TN, DF, DF), lambda i: (0, 0, 0)),
                  pl.BlockSpec((8, DF), lambda i: (0, 0))],
        out_specs=pl.BlockSpec((2000, DF), lambda i: (i, 0)),
        out_shape=jax.ShapeDtypeStruct((N, DF), jnp.float32),
    )(part, nf, ntype2, wa, wb, bn_pad)


def kernel(nf, ef, edge_index, etype, ntype, We, be, Wn, bn):
    src = edge_index[0]
    dst = edge_index[1]

    ws = jnp.transpose(We[:, DE:DE + DF, :], (1, 0, 2)).reshape(DF, TE * DE)
    wd = jnp.transpose(We[:, DE + DF:, :], (1, 0, 2)).reshape(DF, TE * DE)
    wpad = jnp.concatenate([ws, wd], axis=1)
    eye8 = jnp.eye(8, dtype=jnp.float32)
    wbd = jax.vmap(lambda w: jnp.kron(eye8, w))(We[:, :DE, :])
    bpad = jnp.tile(jnp.concatenate(
        [jnp.zeros((3, DE), jnp.float32), be], axis=0).reshape(1, 96), (8, 1))
    wa = Wn[:, :DE, :]
    wb = Wn[:, DE:, :]
    bn_pad = jnp.concatenate(
        [bn, jnp.zeros((8 - TN, DF), jnp.float32)], axis=0)

    pad = EP - E
    idx_s2 = jnp.pad(src * 6 + etype, (0, pad)).reshape(EP // CHUNK, CHUNK)
    idx_d2 = jnp.pad(dst * 6 + 3 + etype, (0, pad)).reshape(EP // CHUNK, CHUNK)
    ef_rs = ef.reshape(ROWS_REAL, 128)
    et8 = etype.reshape(E // 8, 8)
    exp8 = jnp.kron(jnp.eye(8, dtype=jnp.float32),
                    jnp.ones((1, DE), jnp.float32))
    ntype2 = ntype.reshape(N, 1)
    zeros_n = jnp.zeros((N, DE), jnp.float32)

    p = _proj(nf, wpad, bpad)
    p8 = p.reshape(N * 6, DE)

    s = _sc_edge(p8, idx_s2, idx_d2)
    s_rs = s.reshape(ROWS, 128)

    msg = _msg(ef_rs, et8, s_rs, wbd, exp8).reshape(E, DE)

    part = _sc_scatter(msg, dst, zeros_n)
    updated_nf = _node(part, nf, ntype2, wa, wb, bn_pad)

    return (updated_nf, msg)


# speedup vs baseline: 1.8830x; 1.1496x over previous
import functools

import jax
import jax.numpy as jnp
from jax import lax
from jax.experimental import pallas as pl
from jax.experimental.pallas import tpu as pltpu
from jax.experimental.pallas import tpu_sc as plsc

N = 10000
E = 320000
DF = 128
DE = 16
TE = 3
TN = 2

NW = 32
CHUNK = 128
NCHUNK = 80
PW = CHUNK * NCHUNK
EP = NW * PW
ROWS = EP * DE // 128
ROWS_REAL = E * DE // 128

_HIGH = lax.Precision.HIGHEST


def _dot(a, b):
    return lax.dot_general(a, b, (((1,), (0,)), ((), ())),
                           precision=_HIGH, preferred_element_type=jnp.float32)


def _proj_body(nf_ref, w_ref, b_ref, o_ref):
    o_ref[...] = _dot(nf_ref[...], w_ref[...]) + b_ref[0:1, :]


def _proj(nf, wpad, bpad):
    return pl.pallas_call(
        _proj_body,
        grid=(5,),
        in_specs=[pl.BlockSpec((2000, DF), lambda i: (i, 0)),
                  pl.BlockSpec((DF, 96), lambda i: (0, 0)),
                  pl.BlockSpec((8, 96), lambda i: (0, 0))],
        out_specs=pl.BlockSpec((2000, 96), lambda i: (i, 0)),
        out_shape=jax.ShapeDtypeStruct((N, 96), jnp.float32),
    )(nf, wpad, bpad)


MB = 800


def _dot_h(a, b):
    return lax.dot_general(a, b, (((1,), (0,)), ((), ())),
                           preferred_element_type=jnp.float32)


def _msg_body(ef_ref, et_ref, s_ref, wbd_ref, exp_ref, o_ref):
    x = ef_ref[...]
    et8 = et_ref[...]
    acc = s_ref[...]
    for t in range(TE):
        mask = _dot_h((et8 == t).astype(jnp.float32), exp_ref[...])
        acc += mask * _dot_h(x, wbd_ref[t])
    o_ref[...] = jnp.maximum(acc, 0.0)


def _msg(ef_rs, et8, s_rs, wbd, exp8):
    return pl.pallas_call(
        _msg_body,
        grid=(ROWS_REAL // MB,),
        in_specs=[pl.BlockSpec((MB, 128), lambda i: (i, 0)),
                  pl.BlockSpec((MB, 8), lambda i: (i, 0)),
                  pl.BlockSpec((MB, 128), lambda i: (i, 0)),
                  pl.BlockSpec((TE, 128, 128), lambda i: (0, 0, 0)),
                  pl.BlockSpec((8, 128), lambda i: (0, 0))],
        out_specs=pl.BlockSpec((MB, 128), lambda i: (i, 0)),
        out_shape=jax.ShapeDtypeStruct((ROWS_REAL, 128), jnp.float32),
    )(ef_rs, et8, s_rs, wbd, exp8)


G = 8
GROUP = G * CHUNK
NG = PW // GROUP
NCH = E // CHUNK
KMAX = (NCH + NW - 1) // NW


def _sc_edge_body(p_hbm, is_hbm, id_hbm, s_out_hbm,
                  iv, rv, p_sh, sems0, semd0, sems1, semd1):
    semS = (sems0, sems1)
    semD = (semd0, semd1)
    cid = lax.axis_index("c")
    sid = lax.axis_index("s")
    wid = sid * 2 + cid

    if True:
        @pl.when(sid == 0)
        def _():
            pltpu.sync_copy(p_hbm, p_sh)

        plsc.subcore_barrier()

        base = wid * PW
        brow = wid * (PW // CHUNK)

        def load(g, sl):
            pltpu.sync_copy(is_hbm.at[pl.ds(brow + g * G, G)], iv.at[sl, 0])
            pltpu.sync_copy(id_hbm.at[pl.ds(brow + g * G, G)], iv.at[sl, 1])

        def fire(sl):
            for j in range(G):
                pltpu.async_copy(p_sh.at[iv.at[sl, 0, j]],
                                 rv.at[sl, 0, pl.ds(j * CHUNK, CHUNK)],
                                 semS[sl])
                pltpu.async_copy(p_sh.at[iv.at[sl, 1, j]],
                                 rv.at[sl, 1, pl.ds(j * CHUNK, CHUNK)],
                                 semD[sl])

        def wait(sl):
            for j in range(G):
                pltpu.make_async_copy(p_sh.at[iv.at[sl, 0, j]],
                                      rv.at[sl, 0, pl.ds(j * CHUNK, CHUNK)],
                                      semS[sl]).wait()
                pltpu.make_async_copy(p_sh.at[iv.at[sl, 1, j]],
                                      rv.at[sl, 1, pl.ds(j * CHUNK, CHUNK)],
                                      semD[sl]).wait()

        def compute(sl):
            @pl.loop(0, GROUP)
            def _(e):
                rv[sl, 0, e, :] = rv[sl, 0, e, :] + rv[sl, 1, e, :]

        def store(g, sl):
            pltpu.sync_copy(rv.at[sl, 0],
                            s_out_hbm.at[pl.ds(base + g * GROUP, GROUP)])

        load(0, 0)
        fire(0)

        @pl.loop(0, NG, step=2)
        def _(g):
            load(g + 1, 1)
            fire(1)
            wait(0)
            compute(0)
            store(g, 0)

            @pl.when(g + 2 < NG)
            def _():
                load(g + 2, 0)
                fire(0)

            wait(1)
            compute(1)
            store(g + 1, 1)


def _sc_scatter_body(uef_hbm, didx_hbm, zero_hbm, part_hbm,
                     vv, iv2, agg_sh, seml0, seml1):
    cid = lax.axis_index("c")
    sid = lax.axis_index("s")
    wid = sid * 2 + cid

    if True:
        @pl.when(sid == 0)
        def _():
            pltpu.sync_copy(zero_hbm, agg_sh)

        plsc.subcore_barrier()

        sems2 = (seml0, seml1)

        def aload(k, sl):
            off = (wid + k * NW) * CHUNK
            pltpu.async_copy(uef_hbm.at[pl.ds(off, CHUNK)], vv.at[sl],
                             sems2[sl])
            pltpu.async_copy(didx_hbm.at[pl.ds(off, CHUNK)], iv2.at[sl],
                             sems2[sl])

        def wload(k, sl):
            off = (wid + k * NW) * CHUNK
            pltpu.make_async_copy(uef_hbm.at[pl.ds(off, CHUNK)], vv.at[sl],
                                  sems2[sl]).wait()
            pltpu.make_async_copy(didx_hbm.at[pl.ds(off, CHUNK)], iv2.at[sl],
                                  sems2[sl]).wait()

        def sadd(sl):
            pltpu.sync_copy(vv.at[sl], agg_sh.at[iv2.at[sl]], add=True)

        nk = 78 + jnp.where(wid + 78 * NW < NCH, 1, 0)

        aload(0, 0)

        @pl.loop(0, KMAX, step=2)
        def _(k):
            @pl.when(k < nk)
            def _():
                @pl.when(k + 1 < nk)
                def _():
                    aload(k + 1, 1)

                wload(k, 0)
                sadd(0)

                @pl.when(k + 2 < nk)
                def _():
                    aload(k + 2, 0)

                @pl.when(k + 1 < nk)
                def _():
                    wload(k + 1, 1)
                    sadd(1)

        plsc.subcore_barrier()
        rows = N // 16
        pltpu.sync_copy(agg_sh.at[pl.ds(sid * rows, rows)],
                        part_hbm.at[cid, pl.ds(sid * rows, rows)])


def _sc_edge(p8, idx_s2, idx_d2):
    mesh = plsc.VectorSubcoreMesh(core_axis_name="c", subcore_axis_name="s")
    f = pl.kernel(
        _sc_edge_body,
        mesh=mesh,
        compiler_params=pltpu.CompilerParams(use_tc_tiling_on_sc=False),
        out_type=jax.ShapeDtypeStruct((EP, DE), jnp.float32),
        scratch_types=[pltpu.VMEM((2, 2, G, CHUNK), jnp.int32),
                       pltpu.VMEM((2, 2, GROUP, DE), jnp.float32),
                       pltpu.VMEM_SHARED((N * 6, DE), jnp.float32),
                       pltpu.SemaphoreType.DMA,
                       pltpu.SemaphoreType.DMA,
                       pltpu.SemaphoreType.DMA,
                       pltpu.SemaphoreType.DMA],
    )
    return f(p8, idx_s2, idx_d2)


def _sc_scatter(uef, dst, zeros_n):
    mesh = plsc.VectorSubcoreMesh(core_axis_name="c", subcore_axis_name="s")
    f = pl.kernel(
        _sc_scatter_body,
        mesh=mesh,
        compiler_params=pltpu.CompilerParams(use_tc_tiling_on_sc=False),
        out_type=jax.ShapeDtypeStruct((2, N, DE), jnp.float32),
        scratch_types=[pltpu.VMEM((2, CHUNK, DE), jnp.float32),
                       pltpu.VMEM((2, CHUNK), jnp.int32),
                       pltpu.VMEM_SHARED((N, DE), jnp.float32),
                       pltpu.SemaphoreType.DMA,
                       pltpu.SemaphoreType.DMA],
    )
    return f(uef, dst, zeros_n)


def _node_body(part_ref, nf_ref, nt_ref, wa_ref, wb_ref, bn_ref, o_ref):
    agg = part_ref[0] + part_ref[1]
    x = nf_ref[...]
    nt = nt_ref[...]
    y0 = jnp.maximum(_dot(agg, wa_ref[0]) + _dot(x, wb_ref[0]) + bn_ref[0], 0.0)
    y1 = jnp.maximum(_dot(agg, wa_ref[1]) + _dot(x, wb_ref[1]) + bn_ref[1], 0.0)
    o_ref[...] = jnp.where(nt == 0, y0, y1)


def _node(part, nf, ntype2, wa, wb, bn_pad):
    return pl.pallas_call(
        _node_body,
        grid=(5,),
        in_specs=[pl.BlockSpec((2, 2000, DE), lambda i: (0, i, 0)),
                  pl.BlockSpec((2000, DF), lambda i: (i, 0)),
                  pl.BlockSpec((2000, 1), lambda i: (i, 0)),
                  pl.BlockSpec((TN, DE, DF), lambda i: (0, 0, 0)),
                  pl.BlockSpec((TN, DF, DF), lambda i: (0, 0, 0)),
                  pl.BlockSpec((8, DF), lambda i: (0, 0))],
        out_specs=pl.BlockSpec((2000, DF), lambda i: (i, 0)),
        out_shape=jax.ShapeDtypeStruct((N, DF), jnp.float32),
    )(part, nf, ntype2, wa, wb, bn_pad)


def kernel(nf, ef, edge_index, etype, ntype, We, be, Wn, bn):
    src = edge_index[0]
    dst = edge_index[1]

    ws = jnp.transpose(We[:, DE:DE + DF, :], (1, 0, 2)).reshape(DF, TE * DE)
    wd = jnp.transpose(We[:, DE + DF:, :], (1, 0, 2)).reshape(DF, TE * DE)
    wpad = jnp.concatenate([ws, wd], axis=1)
    eye8 = jnp.eye(8, dtype=jnp.float32)
    wbd = jax.vmap(lambda w: jnp.kron(eye8, w))(We[:, :DE, :])
    bpad = jnp.tile(jnp.concatenate(
        [jnp.zeros((3, DE), jnp.float32), be], axis=0).reshape(1, 96), (8, 1))
    wa = Wn[:, :DE, :]
    wb = Wn[:, DE:, :]
    bn_pad = jnp.concatenate(
        [bn, jnp.zeros((8 - TN, DF), jnp.float32)], axis=0)

    pad = EP - E
    idx_s2 = jnp.pad(src * 6 + etype, (0, pad)).reshape(EP // CHUNK, CHUNK)
    idx_d2 = jnp.pad(dst * 6 + 3 + etype, (0, pad)).reshape(EP // CHUNK, CHUNK)
    ef_rs = ef.reshape(ROWS_REAL, 128)
    et8 = etype.reshape(E // 8, 8)
    exp8 = jnp.kron(jnp.eye(8, dtype=jnp.float32),
                    jnp.ones((1, DE), jnp.float32))
    ntype2 = ntype.reshape(N, 1)
    zeros_n = jnp.zeros((N, DE), jnp.float32)

    p = _proj(nf, wpad, bpad)
    p8 = p.reshape(N * 6, DE)

    s = _sc_edge(p8, idx_s2, idx_d2)
    s_rs = s.reshape(ROWS, 128)

    msg = _msg(ef_rs, et8, s_rs, wbd, exp8).reshape(E, DE)

    part = _sc_scatter(msg, dst, zeros_n)
    updated_nf = _node(part, nf, ntype2, wa, wb, bn_pad)

    return (updated_nf, msg)


# speedup vs baseline: 1.8884x; 1.0028x over previous
import functools

import jax
import jax.numpy as jnp
from jax import lax
from jax.experimental import pallas as pl
from jax.experimental.pallas import tpu as pltpu
from jax.experimental.pallas import tpu_sc as plsc

N = 10000
E = 320000
DF = 128
DE = 16
TE = 3
TN = 2

NW = 32
CHUNK = 128
NCHUNK = 80
PW = CHUNK * NCHUNK
EP = NW * PW
ROWS = EP * DE // 128
ROWS_REAL = E * DE // 128

def _dot(a, b):
    return lax.dot_general(a, b, (((1,), (0,)), ((), ())),
                           preferred_element_type=jnp.float32)


def _proj_body(nf_ref, w_ref, b_ref, o_ref):
    o_ref[...] = _dot(nf_ref[...], w_ref[...]) + b_ref[0:1, :]


def _proj(nf, wpad, bpad):
    return pl.pallas_call(
        _proj_body,
        grid=(5,),
        in_specs=[pl.BlockSpec((2000, DF), lambda i: (i, 0)),
                  pl.BlockSpec((DF, 96), lambda i: (0, 0)),
                  pl.BlockSpec((8, 96), lambda i: (0, 0))],
        out_specs=pl.BlockSpec((2000, 96), lambda i: (i, 0)),
        out_shape=jax.ShapeDtypeStruct((N, 96), jnp.float32),
    )(nf, wpad, bpad)


MB = 800


def _dot_h(a, b):
    return lax.dot_general(a, b, (((1,), (0,)), ((), ())),
                           preferred_element_type=jnp.float32)


def _msg_body(ef_ref, et_ref, s_ref, wbd_ref, exp_ref, o_ref):
    x = ef_ref[...]
    et8 = et_ref[...]
    acc = s_ref[...]
    for t in range(TE):
        mask = _dot_h((et8 == t).astype(jnp.float32), exp_ref[...])
        acc += mask * _dot_h(x, wbd_ref[t])
    o_ref[...] = jnp.maximum(acc, 0.0)


def _msg(ef_rs, et8, s_rs, wbd, exp8):
    return pl.pallas_call(
        _msg_body,
        grid=(ROWS_REAL // MB,),
        in_specs=[pl.BlockSpec((MB, 128), lambda i: (i, 0)),
                  pl.BlockSpec((MB, 8), lambda i: (i, 0)),
                  pl.BlockSpec((MB, 128), lambda i: (i, 0)),
                  pl.BlockSpec((TE, 128, 128), lambda i: (0, 0, 0)),
                  pl.BlockSpec((8, 128), lambda i: (0, 0))],
        out_specs=pl.BlockSpec((MB, 128), lambda i: (i, 0)),
        out_shape=jax.ShapeDtypeStruct((ROWS_REAL, 128), jnp.float32),
    )(ef_rs, et8, s_rs, wbd, exp8)


G = 4
GROUP = G * CHUNK
NG = PW // GROUP
SROW = GROUP * DE // 128
NCH = E // CHUNK
KMAX = (NCH + NW - 1) // NW


def _sc_edge_body(p_hbm, is_hbm, id_hbm, s_out_hbm,
                  iv, rv, ov, p_sh, sems0, semd0, sems1, semd1):
    semS = (sems0, sems1)
    semD = (semd0, semd1)
    cid = lax.axis_index("c")
    sid = lax.axis_index("s")
    wid = sid * 2 + cid

    if True:
        @pl.when(sid == 0)
        def _():
            pltpu.sync_copy(p_hbm, p_sh)

        plsc.subcore_barrier()

        base = wid * PW
        brow = wid * (PW // CHUNK)

        def load(g, sl):
            pltpu.sync_copy(is_hbm.at[pl.ds(brow + g * G, G)], iv.at[sl, 0])
            pltpu.sync_copy(id_hbm.at[pl.ds(brow + g * G, G)], iv.at[sl, 1])

        def fire(sl):
            for j in range(G):
                pltpu.async_copy(p_sh.at[iv.at[sl, 0, j]],
                                 rv.at[sl, 0, pl.ds(j * CHUNK, CHUNK)],
                                 semS[sl])
                pltpu.async_copy(p_sh.at[iv.at[sl, 1, j]],
                                 rv.at[sl, 1, pl.ds(j * CHUNK, CHUNK)],
                                 semD[sl])

        def wait(sl):
            for j in range(G):
                pltpu.make_async_copy(p_sh.at[iv.at[sl, 0, j]],
                                      rv.at[sl, 0, pl.ds(j * CHUNK, CHUNK)],
                                      semS[sl]).wait()
                pltpu.make_async_copy(p_sh.at[iv.at[sl, 1, j]],
                                      rv.at[sl, 1, pl.ds(j * CHUNK, CHUNK)],
                                      semD[sl]).wait()

        def compute(sl):
            @pl.loop(0, SROW)
            def _(j):
                for k in range(128 // DE):
                    e = j * (128 // DE) + k
                    ov[sl, j, pl.ds(k * DE, DE)] = (rv[sl, 0, e, :]
                                                    + rv[sl, 1, e, :])

        def store(g, sl):
            pltpu.sync_copy(ov.at[sl],
                            s_out_hbm.at[pl.ds(base // 8 + g * SROW, SROW)])

        load(0, 0)
        fire(0)

        @pl.loop(0, NG, step=2)
        def _(g):
            load(g + 1, 1)
            fire(1)
            wait(0)
            compute(0)
            store(g, 0)

            @pl.when(g + 2 < NG)
            def _():
                load(g + 2, 0)
                fire(0)

            wait(1)
            compute(1)
            store(g + 1, 1)


def _sc_scatter_body(uef_hbm, didx_hbm, zero_hbm, part_hbm,
                     vv, vw, iv2, agg_sh, seml0, seml1):
    cid = lax.axis_index("c")
    sid = lax.axis_index("s")
    wid = sid * 2 + cid

    if True:
        @pl.when(sid == 0)
        def _():
            pltpu.sync_copy(zero_hbm, agg_sh)

        plsc.subcore_barrier()

        sems2 = (seml0, seml1)

        def aload(k, sl):
            c = wid + k * NW
            pltpu.async_copy(uef_hbm.at[pl.ds(c * DE, DE)], vv.at[sl],
                             sems2[sl])
            pltpu.async_copy(didx_hbm.at[pl.ds(c * CHUNK, CHUNK)], iv2.at[sl],
                             sems2[sl])

        def wload(k, sl):
            c = wid + k * NW
            pltpu.make_async_copy(uef_hbm.at[pl.ds(c * DE, DE)], vv.at[sl],
                                  sems2[sl]).wait()
            pltpu.make_async_copy(didx_hbm.at[pl.ds(c * CHUNK, CHUNK)],
                                  iv2.at[sl], sems2[sl]).wait()

        def sadd(sl):
            @pl.loop(0, DE)
            def _(j):
                for kk in range(128 // DE):
                    vw[sl, j * (128 // DE) + kk, :] = \
                        vv[sl, j, pl.ds(kk * DE, DE)]
            pltpu.sync_copy(vw.at[sl], agg_sh.at[iv2.at[sl]], add=True)

        nk = 78 + jnp.where(wid + 78 * NW < NCH, 1, 0)

        aload(0, 0)

        @pl.loop(0, KMAX, step=2)
        def _(k):
            @pl.when(k < nk)
            def _():
                @pl.when(k + 1 < nk)
                def _():
                    aload(k + 1, 1)

                wload(k, 0)
                sadd(0)

                @pl.when(k + 2 < nk)
                def _():
                    aload(k + 2, 0)

                @pl.when(k + 1 < nk)
                def _():
                    wload(k + 1, 1)
                    sadd(1)

        plsc.subcore_barrier()
        rows = N // 16
        pltpu.sync_copy(agg_sh.at[pl.ds(sid * rows, rows)],
                        part_hbm.at[cid, pl.ds(sid * rows, rows)])


def _sc_edge(p8, idx_s2, idx_d2):
    mesh = plsc.VectorSubcoreMesh(core_axis_name="c", subcore_axis_name="s")
    f = pl.kernel(
        _sc_edge_body,
        mesh=mesh,
        compiler_params=pltpu.CompilerParams(use_tc_tiling_on_sc=False),
        out_type=jax.ShapeDtypeStruct((ROWS, 128), jnp.float32),
        scratch_types=[pltpu.VMEM((2, 2, G, CHUNK), jnp.int32),
                       pltpu.VMEM((2, 2, GROUP, DE), jnp.float32),
                       pltpu.VMEM((2, SROW, 128), jnp.float32),
                       pltpu.VMEM_SHARED((N * 6, DE), jnp.float32),
                       pltpu.SemaphoreType.DMA,
                       pltpu.SemaphoreType.DMA,
                       pltpu.SemaphoreType.DMA,
                       pltpu.SemaphoreType.DMA],
    )
    return f(p8, idx_s2, idx_d2)


def _sc_scatter(uef, dst, zeros_n):
    mesh = plsc.VectorSubcoreMesh(core_axis_name="c", subcore_axis_name="s")
    f = pl.kernel(
        _sc_scatter_body,
        mesh=mesh,
        compiler_params=pltpu.CompilerParams(use_tc_tiling_on_sc=False),
        out_type=jax.ShapeDtypeStruct((2, N, DE), jnp.float32),
        scratch_types=[pltpu.VMEM((2, DE, 128), jnp.float32),
                       pltpu.VMEM((2, CHUNK, DE), jnp.float32),
                       pltpu.VMEM((2, CHUNK), jnp.int32),
                       pltpu.VMEM_SHARED((N, DE), jnp.float32),
                       pltpu.SemaphoreType.DMA,
                       pltpu.SemaphoreType.DMA],
    )
    return f(uef, dst, zeros_n)


def _node_body(part_ref, nf_ref, nt_ref, wa_ref, wb_ref, bn_ref, o_ref):
    agg = part_ref[0] + part_ref[1]
    x = nf_ref[...]
    nt = nt_ref[...]
    y0 = jnp.maximum(_dot(agg, wa_ref[0]) + _dot(x, wb_ref[0]) + bn_ref[0], 0.0)
    y1 = jnp.maximum(_dot(agg, wa_ref[1]) + _dot(x, wb_ref[1]) + bn_ref[1], 0.0)
    o_ref[...] = jnp.where(nt == 0, y0, y1)


def _node(part, nf, ntype2, wa, wb, bn_pad):
    return pl.pallas_call(
        _node_body,
        grid=(5,),
        in_specs=[pl.BlockSpec((2, 2000, DE), lambda i: (0, i, 0)),
                  pl.BlockSpec((2000, DF), lambda i: (i, 0)),
                  pl.BlockSpec((2000, 1), lambda i: (i, 0)),
                  pl.BlockSpec((TN, DE, DF), lambda i: (0, 0, 0)),
                  pl.BlockSpec((TN, DF, DF), lambda i: (0, 0, 0)),
                  pl.BlockSpec((8, DF), lambda i: (0, 0))],
        out_specs=pl.BlockSpec((2000, DF), lambda i: (i, 0)),
        out_shape=jax.ShapeDtypeStruct((N, DF), jnp.float32),
    )(part, nf, ntype2, wa, wb, bn_pad)


def kernel(nf, ef, edge_index, etype, ntype, We, be, Wn, bn):
    src = edge_index[0]
    dst = edge_index[1]

    ws = jnp.transpose(We[:, DE:DE + DF, :], (1, 0, 2)).reshape(DF, TE * DE)
    wd = jnp.transpose(We[:, DE + DF:, :], (1, 0, 2)).reshape(DF, TE * DE)
    wpad = jnp.concatenate([ws, wd], axis=1)
    eye8 = jnp.eye(8, dtype=jnp.float32)
    wbd = jax.vmap(lambda w: jnp.kron(eye8, w))(We[:, :DE, :])
    bpad = jnp.tile(jnp.concatenate(
        [jnp.zeros((3, DE), jnp.float32), be], axis=0).reshape(1, 96), (8, 1))
    wa = Wn[:, :DE, :]
    wb = Wn[:, DE:, :]
    bn_pad = jnp.concatenate(
        [bn, jnp.zeros((8 - TN, DF), jnp.float32)], axis=0)

    pad = EP - E
    idx_s2 = jnp.pad(src * 6 + etype, (0, pad)).reshape(EP // CHUNK, CHUNK)
    idx_d2 = jnp.pad(dst * 6 + 3 + etype, (0, pad)).reshape(EP // CHUNK, CHUNK)
    ef_rs = ef.reshape(ROWS_REAL, 128)
    et8 = etype.reshape(E // 8, 8)
    exp8 = jnp.kron(jnp.eye(8, dtype=jnp.float32),
                    jnp.ones((1, DE), jnp.float32))
    ntype2 = ntype.reshape(N, 1)
    zeros_n = jnp.zeros((N, DE), jnp.float32)

    p = _proj(nf, wpad, bpad)
    p8 = p.reshape(N * 6, DE)

    s_rs = _sc_edge(p8, idx_s2, idx_d2)

    msg128 = _msg(ef_rs, et8, s_rs, wbd, exp8)

    part = _sc_scatter(msg128, dst, zeros_n)
    updated_nf = _node(part, nf, ntype2, wa, wb, bn_pad)

    return (updated_nf, msg128.reshape(E, DE))
